# baseline (speedup 1.0000x reference)
import functools

import jax
import jax.numpy as jnp
from jax import lax
from jax.experimental import pallas as pl
from jax.experimental.pallas import tpu as pltpu
from jax.experimental.pallas import tpu_sc as plsc

N = 10000
NP = 10240
E = 320000
EP = 327680
F = 128
B = 64
CHUNK = 128
NCH = EP // CHUNK
ROWS_PER_TILE = NP // 16
AR = 10112
HC = 32
NPH = (NCH // 16) // HC


def _zero_vec(ref, nwords):
    def body(i, _):
        ref[pl.ds(i * 16, 16)] = jnp.zeros((16,), jnp.float32)
        return 0
    lax.fori_loop(0, nwords // 16, body, 0)


def _deg_body(col_hbm, deg_out, cidx, ones_v, zbuf, sem, deg_sh):
    cid = lax.axis_index("c")
    sid = lax.axis_index("s")
    CPD = NCH // 32

    _zero_vec(zbuf, ROWS_PER_TILE)
    def ones_body(i, _):
        ones_v[pl.ds(i * 16, 16)] = jnp.ones((16,), jnp.float32)
        return 0
    lax.fori_loop(0, CHUNK // 16, ones_body, 0)

    pltpu.sync_copy(zbuf, deg_sh.at[pl.ds(sid * ROWS_PER_TILE, ROWS_PER_TILE)])
    plsc.subcore_barrier()

    base = cid * (NCH // 2) + sid * CPD
    pltpu.sync_copy(col_hbm.at[pl.ds(base, CPD)], cidx)
    def fire(j, _):
        pltpu.async_copy(ones_v, deg_sh.at[cidx.at[j]], sem, add=True)
        return 0
    lax.fori_loop(0, CPD, fire, 0)
    def drain(j, _):
        pltpu.make_async_copy(ones_v, deg_sh.at[cidx.at[0]], sem).wait()
        return 0
    lax.fori_loop(0, CPD, drain, 0)

    plsc.subcore_barrier()
    sl = pl.ds(sid * ROWS_PER_TILE, ROWS_PER_TILE)
    pltpu.sync_copy(deg_sh.at[sl], deg_out.at[cid].at[sl])


def _deg_call(col2d):
    mesh = plsc.VectorSubcoreMesh(core_axis_name="c", subcore_axis_name="s")
    fn = pl.kernel(
        _deg_body,
        mesh=mesh,
        out_type=jax.ShapeDtypeStruct((2, NP), jnp.float32),
        scratch_types=[
            pltpu.VMEM((NCH // 32, CHUNK), jnp.int32),
            pltpu.VMEM((CHUNK,), jnp.float32),
            pltpu.VMEM((ROWS_PER_TILE,), jnp.float32),
            pltpu.SemaphoreType.DMA,
            pltpu.VMEM_SHARED((NP,), jnp.float32),
        ],
    )
    return fn(col2d)


def _scat_body(g1, g2, row2, col2, s1, s2,
               ridx0, cidx0, ridx1, cidx1, buf0, buf1,
               gsem0, gsem1, ssem0, ssem1, stg0, stg1,
               acc_sh):
    cid = lax.axis_index("c")
    sid = lax.axis_index("s")
    bufs = (buf0, buf1)
    gsems = (gsem0, gsem1)
    ssems = (ssem0, ssem1)
    ridxs = (ridx0, ridx1)
    cidxs = (cidx0, cidx1)
    stgs = (stg0, stg1)
    CPT = NCH // 16
    ZR = AR // 16
    NZC = ZR // CHUNK

    def zrow(r, _):
        for cc in range(F // 16):
            buf0[r, pl.ds(cc * 16, 16)] = jnp.zeros((16,), jnp.float32)
        return 0
    lax.fori_loop(0, CHUNK, zrow, 0)
    for k in range(NZC):
        pltpu.sync_copy(buf0, acc_sh.at[pl.ds(sid * ZR + k * CHUNK, CHUNK)])
    pltpu.sync_copy(buf0.at[pl.ds(0, ZR - NZC * CHUNK)],
                    acc_sh.at[pl.ds(sid * ZR + NZC * CHUNK, ZR - NZC * CHUNK)])
    TAIL = (NP - AR) // 16

    @pl.when(cid == 0)
    def _():
        pltpu.sync_copy(buf0.at[pl.ds(0, TAIL)],
                        s1.at[pl.ds(AR + sid * TAIL, TAIL)])

    @pl.when(cid == 1)
    def _():
        pltpu.sync_copy(buf0.at[pl.ds(0, TAIL)],
                        s2.at[pl.ds(AR + sid * TAIL, TAIL)])
    plsc.subcore_barrier()

    def run(g_hbm):
        tbase = sid * CPT

        def g_issue(slot, c, q):
            pltpu.async_copy(g_hbm.at[ridxs[slot].at[c]], bufs[q], gsems[q])

        def g_wait(q):
            pltpu.make_async_copy(g_hbm.at[ridxs[0].at[0]], bufs[q],
                                  gsems[q]).wait()

        def s_issue(slot, c, q):
            pltpu.async_copy(bufs[q], acc_sh.at[cidxs[slot].at[c]],
                             ssems[q], add=True)

        def s_wait(q):
            pltpu.make_async_copy(bufs[q], acc_sh.at[cidxs[0].at[0]],
                                  ssems[q]).wait()

        def stg_issue(slot, ph_next):
            base = tbase + ph_next * HC
            pltpu.async_copy(row2.at[pl.ds(base, HC)], ridxs[slot], stgs[slot])
            pltpu.async_copy(col2.at[pl.ds(base, HC)], cidxs[slot], stgs[slot])

        def stg_wait(slot):
            pltpu.make_async_copy(row2.at[pl.ds(tbase, HC)], ridxs[slot],
                                  stgs[slot]).wait()
            pltpu.make_async_copy(col2.at[pl.ds(tbase, HC)], cidxs[slot],
                                  stgs[slot]).wait()

        pltpu.sync_copy(row2.at[pl.ds(tbase, HC)], ridxs[0])
        pltpu.sync_copy(col2.at[pl.ds(tbase, HC)], cidxs[0])
        g_issue(0, 0, 0)

        for ph in range(NPH):
            slot = ph % 2
            nslot = (ph + 1) % 2
            last_ph = ph == NPH - 1

            def blk(b, _, slot=slot, nslot=nslot, ph=ph, last_ph=last_ph):
                for j in range(16):
                    lc = b * 16 + j
                    q = j % 2
                    nq = (j + 1) % 2

                    if ph == 0:
                        @pl.when(lc >= 1)
                        def _():
                            s_wait(nq)
                    else:
                        s_wait(nq)

                    if not last_ph and j == 0:
                        @pl.when(b == 0)
                        def _():
                            stg_issue(nslot, ph + 1)

                    @pl.when(lc + 1 < HC)
                    def _():
                        g_issue(slot, lc + 1, nq)

                    if not last_ph and j == 15:
                        @pl.when(b == HC // 16 - 1)
                        def _():
                            stg_wait(nslot)
                            g_issue(nslot, 0, nq)

                    g_wait(q)
                    s_issue(slot, lc, q)
                return 0
            lax.fori_loop(0, HC // 16, blk, 0)
        s_wait((HC - 1) % 2)

    @pl.when(cid == 0)
    def _():
        run(g1)

    @pl.when(cid == 1)
    def _():
        run(g2)

    plsc.subcore_barrier()

    def wout(s_hbm):
        for k in range(4):
            sl = pl.ds(sid * ZR + k * CHUNK, CHUNK)
            pltpu.sync_copy(acc_sh.at[sl], s_hbm.at[sl])
        sl = pl.ds(sid * ZR + 4 * CHUNK, ZR - 4 * CHUNK)
        pltpu.sync_copy(acc_sh.at[sl], s_hbm.at[sl])

    @pl.when(cid == 0)
    def _():
        wout(s1)

    @pl.when(cid == 1)
    def _():
        wout(s2)


def _scat_call(g1, g2, row2d, col2d):
    mesh = plsc.VectorSubcoreMesh(core_axis_name="c", subcore_axis_name="s")
    fn = pl.kernel(
        _scat_body,
        mesh=mesh,
        out_type=[jax.ShapeDtypeStruct((NP, F), jnp.float32),
                  jax.ShapeDtypeStruct((NP, F), jnp.float32)],
        scratch_types=[
            pltpu.VMEM((HC, CHUNK), jnp.int32),
            pltpu.VMEM((HC, CHUNK), jnp.int32),
            pltpu.VMEM((HC, CHUNK), jnp.int32),
            pltpu.VMEM((HC, CHUNK), jnp.int32),
            pltpu.VMEM((CHUNK, F), jnp.float32),
            pltpu.VMEM((CHUNK, F), jnp.float32),
            pltpu.SemaphoreType.DMA,
            pltpu.SemaphoreType.DMA,
            pltpu.SemaphoreType.DMA,
            pltpu.SemaphoreType.DMA,
            pltpu.SemaphoreType.DMA,
            pltpu.SemaphoreType.DMA,
            pltpu.VMEM_SHARED((AR, F), jnp.float32),
        ],
    )
    return fn(g1, g2, row2d, col2d)


def _tc1_body(x_ref, w1_ref, w2_ref, deg_ref, g1_ref, g2_ref, dinv_ref):
    xb = x_ref[...]
    d = deg_ref[0] + deg_ref[1] + 1.0
    dinv = lax.rsqrt(d)
    h1 = jnp.dot(xb, w1_ref[...], preferred_element_type=jnp.float32)
    h2 = jnp.dot(xb, w2_ref[...], preferred_element_type=jnp.float32)
    g1_ref[...] = h1 * dinv
    g2_ref[...] = h2 * dinv
    dinv_ref[...] = dinv


def _tc1_call(x_p, W1, W2, deg3):
    nb = NP // 1024
    return pl.pallas_call(
        _tc1_body,
        grid=(nb,),
        in_specs=[
            pl.BlockSpec((1024, F), lambda i: (i, 0)),
            pl.BlockSpec((F, F), lambda i: (0, 0)),
            pl.BlockSpec((F, F), lambda i: (0, 0)),
            pl.BlockSpec((2, 1024, 1), lambda i: (0, i, 0)),
        ],
        out_specs=[
            pl.BlockSpec((1024, F), lambda i: (i, 0)),
            pl.BlockSpec((1024, F), lambda i: (i, 0)),
            pl.BlockSpec((1024, 1), lambda i: (i, 0)),
        ],
        out_shape=[jax.ShapeDtypeStruct((NP, F), jnp.float32),
                   jax.ShapeDtypeStruct((NP, F), jnp.float32),
                   jax.ShapeDtypeStruct((NP, 1), jnp.float32)],
    )(x_p, W1, W2, deg3)


def _tc2_body(s1_ref, s2_ref, g1_ref, g2_ref, dinv_ref, b1_ref, b2_ref,
              batch_ref, eps_ref, z_ref, mu_ref, lv_ref,
              p1_acc, p2_acc, cnt_acc):
    i = pl.program_id(0)

    @pl.when(i == 0)
    def _():
        p1_acc[...] = jnp.zeros_like(p1_acc)
        p2_acc[...] = jnp.zeros_like(p2_acc)
        cnt_acc[...] = jnp.zeros_like(cnt_acc)

    dinv = dinv_ref[...]
    r1 = jnp.maximum(dinv * (s1_ref[...] + g1_ref[...]) + b1_ref[...], 0.0)
    r2 = jnp.maximum(dinv * (s2_ref[...] + g2_ref[...]) + b2_ref[...], 0.0)

    bt = batch_ref[...]
    seg = lax.broadcasted_iota(jnp.int32, (1024, B), 1)
    oh = jnp.where(bt == seg, 1.0, 0.0).astype(jnp.float32)
    dn = (((0,), (0,)), ((), ()))
    p1_acc[...] += lax.dot_general(oh, r1, dn, preferred_element_type=jnp.float32)
    p2_acc[...] += lax.dot_general(oh, r2, dn, preferred_element_type=jnp.float32)
    cnt_acc[...] += lax.dot_general(oh, jnp.ones((1024, F), jnp.float32), dn,
                                    preferred_element_type=jnp.float32)

    @pl.when(i == pl.num_programs(0) - 1)
    def _():
        cnt = jnp.maximum(cnt_acc[...], 1.0)
        mu = p1_acc[...] / cnt
        lv = p2_acc[...] / cnt
        mu_ref[...] = mu
        lv_ref[...] = lv
        z_ref[...] = eps_ref[...] * jnp.exp(lv) + mu


def _tc2_call(s1, s2, g1, g2, dinv, b1, b2, batch2d, eps):
    nb = NP // 1024
    blk = lambda i: (i, 0)
    return pl.pallas_call(
        _tc2_body,
        grid=(nb,),
        in_specs=[
            pl.BlockSpec((1024, F), blk),
            pl.BlockSpec((1024, F), blk),
            pl.BlockSpec((1024, F), blk),
            pl.BlockSpec((1024, F), blk),
            pl.BlockSpec((1024, 1), blk),
            pl.BlockSpec((1, F), lambda i: (0, 0)),
            pl.BlockSpec((1, F), lambda i: (0, 0)),
            pl.BlockSpec((1024, 1), blk),
            pl.BlockSpec((B, F), lambda i: (0, 0)),
        ],
        out_specs=[
            pl.BlockSpec((B, F), lambda i: (0, 0)),
            pl.BlockSpec((B, F), lambda i: (0, 0)),
            pl.BlockSpec((B, F), lambda i: (0, 0)),
        ],
        out_shape=[jax.ShapeDtypeStruct((B, F), jnp.float32),
                   jax.ShapeDtypeStruct((B, F), jnp.float32),
                   jax.ShapeDtypeStruct((B, F), jnp.float32)],
        scratch_shapes=[
            pltpu.VMEM((B, F), jnp.float32),
            pltpu.VMEM((B, F), jnp.float32),
            pltpu.VMEM((B, F), jnp.float32),
        ],
    )(s1, s2, g1, g2, dinv, b1, b2, batch2d, eps)


def kernel(x, edge_index, batch, W1, b1, W2, b2):
    row = edge_index[0]
    col = edge_index[1]
    pad_idx = (N + (jnp.arange(EP - E, dtype=jnp.int32) % 40))
    row_p = jnp.concatenate([row, pad_idx]).reshape(NCH, CHUNK)
    col_p = jnp.concatenate([col, pad_idx]).reshape(NCH, CHUNK)
    x_p = jnp.pad(x, ((0, NP - N), (0, 0)))
    batch2d = jnp.pad(batch, (0, NP - N), constant_values=B).reshape(NP, 1)
    eps = jax.random.normal(jax.random.key(1), (B, F), dtype=jnp.float32)

    deg2 = _deg_call(col_p)
    deg3 = deg2.reshape(2, NP, 1)
    g1, g2, dinv = _tc1_call(x_p, W1, W2, deg3)
    s1, s2 = _scat_call(g1, g2, row_p, col_p)
    z, mu, lv = _tc2_call(s1, s2, g1, g2, dinv,
                          b1.reshape(1, F), b2.reshape(1, F), batch2d, eps)
    return (z, mu, lv)


# speedup vs baseline: 1.0057x; 1.0057x over previous
import functools

import jax
import jax.numpy as jnp
from jax import lax
from jax.experimental import pallas as pl
from jax.experimental.pallas import tpu as pltpu
from jax.experimental.pallas import tpu_sc as plsc

N = 10000
NP = 10240
E = 320000
EP = 327680
F = 128
B = 64
CHUNK = 64
NCH = EP // CHUNK
ROWS_PER_TILE = NP // 16
AR = 10112
HC = 32
NPH = (NCH // 16) // HC


def _zero_vec(ref, nwords):
    def body(i, _):
        ref[pl.ds(i * 16, 16)] = jnp.zeros((16,), jnp.float32)
        return 0
    lax.fori_loop(0, nwords // 16, body, 0)


def _deg_body(col_hbm, deg_out, cidx, ones_v, zbuf, sem, deg_sh):
    cid = lax.axis_index("c")
    sid = lax.axis_index("s")
    CPD = NCH // 32

    _zero_vec(zbuf, ROWS_PER_TILE)
    def ones_body(i, _):
        ones_v[pl.ds(i * 16, 16)] = jnp.ones((16,), jnp.float32)
        return 0
    lax.fori_loop(0, CHUNK // 16, ones_body, 0)

    pltpu.sync_copy(zbuf, deg_sh.at[pl.ds(sid * ROWS_PER_TILE, ROWS_PER_TILE)])
    plsc.subcore_barrier()

    base = cid * (NCH // 2) + sid * CPD
    pltpu.sync_copy(col_hbm.at[pl.ds(base, CPD)], cidx)
    def fire(j, _):
        pltpu.async_copy(ones_v, deg_sh.at[cidx.at[j]], sem, add=True)
        return 0
    lax.fori_loop(0, CPD, fire, 0)
    def drain(j, _):
        pltpu.make_async_copy(ones_v, deg_sh.at[cidx.at[0]], sem).wait()
        return 0
    lax.fori_loop(0, CPD, drain, 0)

    plsc.subcore_barrier()
    sl = pl.ds(sid * ROWS_PER_TILE, ROWS_PER_TILE)
    pltpu.sync_copy(deg_sh.at[sl], deg_out.at[cid].at[sl])


def _deg_call(col2d):
    mesh = plsc.VectorSubcoreMesh(core_axis_name="c", subcore_axis_name="s")
    fn = pl.kernel(
        _deg_body,
        mesh=mesh,
        out_type=jax.ShapeDtypeStruct((2, NP), jnp.float32),
        scratch_types=[
            pltpu.VMEM((NCH // 32, CHUNK), jnp.int32),
            pltpu.VMEM((CHUNK,), jnp.float32),
            pltpu.VMEM((ROWS_PER_TILE,), jnp.float32),
            pltpu.SemaphoreType.DMA,
            pltpu.VMEM_SHARED((NP,), jnp.float32),
        ],
    )
    return fn(col2d)


def _scat_body(g1, g2, row2, col2, s1, s2,
               ridx0, cidx0, ridx1, cidx1, buf0, buf1, buf2, buf3,
               gsem0, gsem1, gsem2, gsem3, ssem0, ssem1, ssem2, ssem3,
               stg0, stg1, acc_sh):
    cid = lax.axis_index("c")
    sid = lax.axis_index("s")
    bufs = (buf0, buf1, buf2, buf3)
    gsems = (gsem0, gsem1, gsem2, gsem3)
    ssems = (ssem0, ssem1, ssem2, ssem3)
    ridxs = (ridx0, ridx1)
    cidxs = (cidx0, cidx1)
    stgs = (stg0, stg1)
    CPT = NCH // 16
    ZR = AR // 16
    NZC = ZR // CHUNK

    def zrow(r, _):
        for cc in range(F // 16):
            buf0[r, pl.ds(cc * 16, 16)] = jnp.zeros((16,), jnp.float32)
        return 0
    lax.fori_loop(0, CHUNK, zrow, 0)
    for k in range(NZC):
        pltpu.sync_copy(buf0, acc_sh.at[pl.ds(sid * ZR + k * CHUNK, CHUNK)])
    pltpu.sync_copy(buf0.at[pl.ds(0, ZR - NZC * CHUNK)],
                    acc_sh.at[pl.ds(sid * ZR + NZC * CHUNK, ZR - NZC * CHUNK)])
    TAIL = (NP - AR) // 16

    @pl.when(cid == 0)
    def _():
        pltpu.sync_copy(buf0.at[pl.ds(0, TAIL)],
                        s1.at[pl.ds(AR + sid * TAIL, TAIL)])

    @pl.when(cid == 1)
    def _():
        pltpu.sync_copy(buf0.at[pl.ds(0, TAIL)],
                        s2.at[pl.ds(AR + sid * TAIL, TAIL)])
    plsc.subcore_barrier()

    def run(g_hbm):
        tbase = sid * CPT

        def g_issue(slot, c, q):
            pltpu.async_copy(g_hbm.at[ridxs[slot].at[c]], bufs[q], gsems[q])

        def g_wait(q):
            pltpu.make_async_copy(g_hbm.at[ridxs[0].at[0]], bufs[q],
                                  gsems[q]).wait()

        def s_issue(slot, c, q):
            pltpu.async_copy(bufs[q], acc_sh.at[cidxs[slot].at[c]],
                             ssems[q], add=True)

        def s_wait(q):
            pltpu.make_async_copy(bufs[q], acc_sh.at[cidxs[0].at[0]],
                                  ssems[q]).wait()

        def stg_issue(slot, ph_next):
            base = tbase + ph_next * HC
            pltpu.async_copy(row2.at[pl.ds(base, HC)], ridxs[slot], stgs[slot])
            pltpu.async_copy(col2.at[pl.ds(base, HC)], cidxs[slot], stgs[slot])

        def stg_wait(slot):
            pltpu.make_async_copy(row2.at[pl.ds(tbase, HC)], ridxs[slot],
                                  stgs[slot]).wait()
            pltpu.make_async_copy(col2.at[pl.ds(tbase, HC)], cidxs[slot],
                                  stgs[slot]).wait()

        pltpu.sync_copy(row2.at[pl.ds(tbase, HC)], ridxs[0])
        pltpu.sync_copy(col2.at[pl.ds(tbase, HC)], cidxs[0])
        g_issue(0, 0, 0)
        g_issue(0, 1, 1)

        for ph in range(NPH):
            slot = ph % 2
            nslot = (ph + 1) % 2
            last_ph = ph == NPH - 1

            def blk(b, _, slot=slot, nslot=nslot, ph=ph, last_ph=last_ph):
                for j in range(16):
                    lc = b * 16 + j
                    q = j % 4
                    q2 = (j + 2) % 4

                    if ph == 0:
                        @pl.when(lc >= 2)
                        def _():
                            s_wait(q2)
                    else:
                        s_wait(q2)

                    if not last_ph and j == 1:
                        @pl.when(b == 0)
                        def _():
                            stg_issue(nslot, ph + 1)

                    @pl.when(lc + 2 < HC)
                    def _():
                        g_issue(slot, lc + 2, q2)

                    if not last_ph and j == 14:
                        @pl.when(b == HC // 16 - 1)
                        def _():
                            stg_wait(nslot)
                            g_issue(nslot, 0, q2)

                    if not last_ph and j == 15:
                        @pl.when(b == HC // 16 - 1)
                        def _():
                            g_issue(nslot, 1, q2)

                    g_wait(q)
                    s_issue(slot, lc, q)
                return 0
            lax.fori_loop(0, HC // 16, blk, 0)
        s_wait((HC - 2) % 4)
        s_wait((HC - 1) % 4)

    @pl.when(cid == 0)
    def _():
        run(g1)

    @pl.when(cid == 1)
    def _():
        run(g2)

    plsc.subcore_barrier()

    def wout(s_hbm):
        for k in range(4):
            sl = pl.ds(sid * ZR + k * CHUNK, CHUNK)
            pltpu.sync_copy(acc_sh.at[sl], s_hbm.at[sl])
        sl = pl.ds(sid * ZR + 4 * CHUNK, ZR - 4 * CHUNK)
        pltpu.sync_copy(acc_sh.at[sl], s_hbm.at[sl])

    @pl.when(cid == 0)
    def _():
        wout(s1)

    @pl.when(cid == 1)
    def _():
        wout(s2)


def _scat_call(g1, g2, row2d, col2d):
    mesh = plsc.VectorSubcoreMesh(core_axis_name="c", subcore_axis_name="s")
    fn = pl.kernel(
        _scat_body,
        mesh=mesh,
        out_type=[jax.ShapeDtypeStruct((NP, F), jnp.float32),
                  jax.ShapeDtypeStruct((NP, F), jnp.float32)],
        scratch_types=[
            pltpu.VMEM((HC, CHUNK), jnp.int32),
            pltpu.VMEM((HC, CHUNK), jnp.int32),
            pltpu.VMEM((HC, CHUNK), jnp.int32),
            pltpu.VMEM((HC, CHUNK), jnp.int32),
            pltpu.VMEM((CHUNK, F), jnp.float32),
            pltpu.VMEM((CHUNK, F), jnp.float32),
            pltpu.VMEM((CHUNK, F), jnp.float32),
            pltpu.VMEM((CHUNK, F), jnp.float32),
            pltpu.SemaphoreType.DMA,
            pltpu.SemaphoreType.DMA,
            pltpu.SemaphoreType.DMA,
            pltpu.SemaphoreType.DMA,
            pltpu.SemaphoreType.DMA,
            pltpu.SemaphoreType.DMA,
            pltpu.SemaphoreType.DMA,
            pltpu.SemaphoreType.DMA,
            pltpu.SemaphoreType.DMA,
            pltpu.SemaphoreType.DMA,
            pltpu.VMEM_SHARED((AR, F), jnp.float32),
        ],
    )
    return fn(g1, g2, row2d, col2d)


def _tc1_body(x_ref, w1_ref, w2_ref, deg_ref, g1_ref, g2_ref, dinv_ref):
    xb = x_ref[...]
    d = deg_ref[0] + deg_ref[1] + 1.0
    dinv = lax.rsqrt(d)
    h1 = jnp.dot(xb, w1_ref[...], preferred_element_type=jnp.float32)
    h2 = jnp.dot(xb, w2_ref[...], preferred_element_type=jnp.float32)
    g1_ref[...] = h1 * dinv
    g2_ref[...] = h2 * dinv
    dinv_ref[...] = dinv


def _tc1_call(x_p, W1, W2, deg3):
    nb = NP // 1024
    return pl.pallas_call(
        _tc1_body,
        grid=(nb,),
        in_specs=[
            pl.BlockSpec((1024, F), lambda i: (i, 0)),
            pl.BlockSpec((F, F), lambda i: (0, 0)),
            pl.BlockSpec((F, F), lambda i: (0, 0)),
            pl.BlockSpec((2, 1024, 1), lambda i: (0, i, 0)),
        ],
        out_specs=[
            pl.BlockSpec((1024, F), lambda i: (i, 0)),
            pl.BlockSpec((1024, F), lambda i: (i, 0)),
            pl.BlockSpec((1024, 1), lambda i: (i, 0)),
        ],
        out_shape=[jax.ShapeDtypeStruct((NP, F), jnp.float32),
                   jax.ShapeDtypeStruct((NP, F), jnp.float32),
                   jax.ShapeDtypeStruct((NP, 1), jnp.float32)],
    )(x_p, W1, W2, deg3)


def _tc2_body(s1_ref, s2_ref, g1_ref, g2_ref, dinv_ref, b1_ref, b2_ref,
              batch_ref, eps_ref, z_ref, mu_ref, lv_ref,
              p1_acc, p2_acc, cnt_acc):
    i = pl.program_id(0)

    @pl.when(i == 0)
    def _():
        p1_acc[...] = jnp.zeros_like(p1_acc)
        p2_acc[...] = jnp.zeros_like(p2_acc)
        cnt_acc[...] = jnp.zeros_like(cnt_acc)

    dinv = dinv_ref[...]
    r1 = jnp.maximum(dinv * (s1_ref[...] + g1_ref[...]) + b1_ref[...], 0.0)
    r2 = jnp.maximum(dinv * (s2_ref[...] + g2_ref[...]) + b2_ref[...], 0.0)

    bt = batch_ref[...]
    seg = lax.broadcasted_iota(jnp.int32, (1024, B), 1)
    oh = jnp.where(bt == seg, 1.0, 0.0).astype(jnp.float32)
    dn = (((0,), (0,)), ((), ()))
    p1_acc[...] += lax.dot_general(oh, r1, dn, preferred_element_type=jnp.float32)
    p2_acc[...] += lax.dot_general(oh, r2, dn, preferred_element_type=jnp.float32)
    cnt_acc[...] += lax.dot_general(oh, jnp.ones((1024, F), jnp.float32), dn,
                                    preferred_element_type=jnp.float32)

    @pl.when(i == pl.num_programs(0) - 1)
    def _():
        cnt = jnp.maximum(cnt_acc[...], 1.0)
        mu = p1_acc[...] / cnt
        lv = p2_acc[...] / cnt
        mu_ref[...] = mu
        lv_ref[...] = lv
        z_ref[...] = eps_ref[...] * jnp.exp(lv) + mu


def _tc2_call(s1, s2, g1, g2, dinv, b1, b2, batch2d, eps):
    nb = NP // 1024
    blk = lambda i: (i, 0)
    return pl.pallas_call(
        _tc2_body,
        grid=(nb,),
        in_specs=[
            pl.BlockSpec((1024, F), blk),
            pl.BlockSpec((1024, F), blk),
            pl.BlockSpec((1024, F), blk),
            pl.BlockSpec((1024, F), blk),
            pl.BlockSpec((1024, 1), blk),
            pl.BlockSpec((1, F), lambda i: (0, 0)),
            pl.BlockSpec((1, F), lambda i: (0, 0)),
            pl.BlockSpec((1024, 1), blk),
            pl.BlockSpec((B, F), lambda i: (0, 0)),
        ],
        out_specs=[
            pl.BlockSpec((B, F), lambda i: (0, 0)),
            pl.BlockSpec((B, F), lambda i: (0, 0)),
            pl.BlockSpec((B, F), lambda i: (0, 0)),
        ],
        out_shape=[jax.ShapeDtypeStruct((B, F), jnp.float32),
                   jax.ShapeDtypeStruct((B, F), jnp.float32),
                   jax.ShapeDtypeStruct((B, F), jnp.float32)],
        scratch_shapes=[
            pltpu.VMEM((B, F), jnp.float32),
            pltpu.VMEM((B, F), jnp.float32),
            pltpu.VMEM((B, F), jnp.float32),
        ],
    )(s1, s2, g1, g2, dinv, b1, b2, batch2d, eps)


def kernel(x, edge_index, batch, W1, b1, W2, b2):
    row = edge_index[0]
    col = edge_index[1]
    pad_idx = (N + (jnp.arange(EP - E, dtype=jnp.int32) % 40))
    row_p = jnp.concatenate([row, pad_idx]).reshape(NCH, CHUNK)
    col_p = jnp.concatenate([col, pad_idx]).reshape(NCH, CHUNK)
    x_p = jnp.pad(x, ((0, NP - N), (0, 0)))
    batch2d = jnp.pad(batch, (0, NP - N), constant_values=B).reshape(NP, 1)
    eps = jax.random.normal(jax.random.key(1), (B, F), dtype=jnp.float32)

    deg2 = _deg_call(col_p)
    deg3 = deg2.reshape(2, NP, 1)
    g1, g2, dinv = _tc1_call(x_p, W1, W2, deg3)
    s1, s2 = _scat_call(g1, g2, row_p, col_p)
    z, mu, lv = _tc2_call(s1, s2, g1, g2, dinv,
                          b1.reshape(1, F), b2.reshape(1, F), batch2d, eps)
    return (z, mu, lv)


# speedup vs baseline: 1.0073x; 1.0016x over previous
import jax
import jax.numpy as jnp
from jax import lax
from jax.experimental import pallas as pl
from jax.experimental.pallas import tpu as pltpu
from jax.experimental.pallas import tpu_sc as plsc

N = 10000
NP = 10240
E = 320000
EP = 327680
F = 128
B = 64
CHUNK = 64
NCH = EP // CHUNK
ROWS_PER_TILE = NP // 16
AR = 10112
HC = 32
NPH = (NCH // 16) // HC


def _zero_vec(ref, nwords):
    def body(i, _):
        ref[pl.ds(i * 16, 16)] = jnp.zeros((16,), jnp.float32)
        return 0
    lax.fori_loop(0, nwords // 16, body, 0)


def _deg_body(col_hbm, deg_out, cidx, ones_v, zbuf, sem, deg_sh):
    cid = lax.axis_index("c")
    sid = lax.axis_index("s")
    CPD = NCH // 32

    _zero_vec(zbuf, ROWS_PER_TILE)
    def ones_body(i, _):
        ones_v[pl.ds(i * 16, 16)] = jnp.ones((16,), jnp.float32)
        return 0
    lax.fori_loop(0, CHUNK // 16, ones_body, 0)

    pltpu.sync_copy(zbuf, deg_sh.at[pl.ds(sid * ROWS_PER_TILE, ROWS_PER_TILE)])
    plsc.subcore_barrier()

    base = cid * (NCH // 2) + sid * CPD
    pltpu.sync_copy(col_hbm.at[pl.ds(base, CPD)], cidx)
    def fire(j, _):
        pltpu.async_copy(ones_v, deg_sh.at[cidx.at[j]], sem, add=True)
        return 0
    lax.fori_loop(0, CPD, fire, 0)
    def drain(j, _):
        pltpu.make_async_copy(ones_v, deg_sh.at[cidx.at[0]], sem).wait()
        return 0
    lax.fori_loop(0, CPD, drain, 0)

    plsc.subcore_barrier()
    sl = pl.ds(sid * ROWS_PER_TILE, ROWS_PER_TILE)
    pltpu.sync_copy(deg_sh.at[sl], deg_out.at[cid].at[sl])


def _deg_call(col2d):
    mesh = plsc.VectorSubcoreMesh(core_axis_name="c", subcore_axis_name="s")
    fn = pl.kernel(
        _deg_body,
        mesh=mesh,
        out_type=jax.ShapeDtypeStruct((2, NP), jnp.float32),
        scratch_types=[
            pltpu.VMEM((NCH // 32, CHUNK), jnp.int32),
            pltpu.VMEM((CHUNK,), jnp.float32),
            pltpu.VMEM((ROWS_PER_TILE,), jnp.float32),
            pltpu.SemaphoreType.DMA,
            pltpu.VMEM_SHARED((NP,), jnp.float32),
        ],
    )
    return fn(col2d)


def _scat_body(g1, g2, row2, col2, s1, s2,
               ridx0, cidx0, ridx1, cidx1, buf0, buf1, buf2, buf3,
               gsem0, gsem1, gsem2, gsem3, ssem0, ssem1, ssem2, ssem3,
               stg0, stg1, acc_sh):
    cid = lax.axis_index("c")
    sid = lax.axis_index("s")
    bufs = (buf0, buf1, buf2, buf3)
    gsems = (gsem0, gsem1, gsem2, gsem3)
    ssems = (ssem0, ssem1, ssem2, ssem3)
    ridxs = (ridx0, ridx1)
    cidxs = (cidx0, cidx1)
    stgs = (stg0, stg1)
    CPT = NCH // 16
    ZR = AR // 16
    NZC = ZR // CHUNK

    def zrow(r, _):
        for cc in range(F // 16):
            buf0[r, pl.ds(cc * 16, 16)] = jnp.zeros((16,), jnp.float32)
        return 0
    lax.fori_loop(0, CHUNK, zrow, 0)
    for k in range(NZC):
        pltpu.sync_copy(buf0, acc_sh.at[pl.ds(sid * ZR + k * CHUNK, CHUNK)])
    pltpu.sync_copy(buf0.at[pl.ds(0, ZR - NZC * CHUNK)],
                    acc_sh.at[pl.ds(sid * ZR + NZC * CHUNK, ZR - NZC * CHUNK)])
    TAIL = (NP - AR) // 16

    @pl.when(cid == 0)
    def _():
        pltpu.sync_copy(buf0.at[pl.ds(0, TAIL)],
                        s1.at[pl.ds(AR + sid * TAIL, TAIL)])

    @pl.when(cid == 1)
    def _():
        pltpu.sync_copy(buf0.at[pl.ds(0, TAIL)],
                        s2.at[pl.ds(AR + sid * TAIL, TAIL)])
    plsc.subcore_barrier()

    def run(g_hbm):
        tbase = sid * CPT

        def g_issue(slot, c, q):
            pltpu.async_copy(g_hbm.at[ridxs[slot].at[c]], bufs[q], gsems[q])

        def g_wait(q):
            pltpu.make_async_copy(g_hbm.at[ridxs[0].at[0]], bufs[q],
                                  gsems[q]).wait()

        def s_issue(slot, c, q):
            pltpu.async_copy(bufs[q], acc_sh.at[cidxs[slot].at[c]],
                             ssems[q], add=True)

        def s_wait(q):
            pltpu.make_async_copy(bufs[q], acc_sh.at[cidxs[0].at[0]],
                                  ssems[q]).wait()

        def stg_issue(slot, ph_next):
            base = tbase + ph_next * HC
            pltpu.async_copy(row2.at[pl.ds(base, HC)], ridxs[slot], stgs[slot])
            pltpu.async_copy(col2.at[pl.ds(base, HC)], cidxs[slot], stgs[slot])

        def stg_wait(slot):
            pltpu.make_async_copy(row2.at[pl.ds(tbase, HC)], ridxs[slot],
                                  stgs[slot]).wait()
            pltpu.make_async_copy(col2.at[pl.ds(tbase, HC)], cidxs[slot],
                                  stgs[slot]).wait()

        pltpu.sync_copy(row2.at[pl.ds(tbase, HC)], ridxs[0])
        pltpu.sync_copy(col2.at[pl.ds(tbase, HC)], cidxs[0])
        g_issue(0, 0, 0)
        g_issue(0, 1, 1)

        for ph in range(NPH):
            slot = ph % 2
            nslot = (ph + 1) % 2
            last_ph = ph == NPH - 1

            def blk(b, _, slot=slot, nslot=nslot, ph=ph, last_ph=last_ph):
                for j in range(16):
                    lc = b * 16 + j
                    q = j % 4
                    q2 = (j + 2) % 4

                    if ph == 0:
                        @pl.when(lc >= 2)
                        def _():
                            s_wait(q2)
                    else:
                        s_wait(q2)

                    if not last_ph and j == 1:
                        @pl.when(b == 0)
                        def _():
                            stg_issue(nslot, ph + 1)

                    @pl.when(lc + 2 < HC)
                    def _():
                        g_issue(slot, lc + 2, q2)

                    if not last_ph and j == 14:
                        @pl.when(b == HC // 16 - 1)
                        def _():
                            stg_wait(nslot)
                            g_issue(nslot, 0, q2)

                    if not last_ph and j == 15:
                        @pl.when(b == HC // 16 - 1)
                        def _():
                            g_issue(nslot, 1, q2)

                    g_wait(q)
                    s_issue(slot, lc, q)
                return 0
            lax.fori_loop(0, HC // 16, blk, 0)
        s_wait((HC - 2) % 4)
        s_wait((HC - 1) % 4)

    @pl.when(cid == 0)
    def _():
        run(g1)

    @pl.when(cid == 1)
    def _():
        run(g2)

    plsc.subcore_barrier()

    def wout(s_hbm):
        for k in range(4):
            sl = pl.ds(sid * ZR + k * CHUNK, CHUNK)
            pltpu.sync_copy(acc_sh.at[sl], s_hbm.at[sl])
        sl = pl.ds(sid * ZR + 4 * CHUNK, ZR - 4 * CHUNK)
        pltpu.sync_copy(acc_sh.at[sl], s_hbm.at[sl])

    @pl.when(cid == 0)
    def _():
        wout(s1)

    @pl.when(cid == 1)
    def _():
        wout(s2)


def _scat_call(g1, g2, row2d, col2d):
    mesh = plsc.VectorSubcoreMesh(core_axis_name="c", subcore_axis_name="s")
    fn = pl.kernel(
        _scat_body,
        mesh=mesh,
        out_type=[jax.ShapeDtypeStruct((NP, F), jnp.float32),
                  jax.ShapeDtypeStruct((NP, F), jnp.float32)],
        scratch_types=[
            pltpu.VMEM((HC, CHUNK), jnp.int32),
            pltpu.VMEM((HC, CHUNK), jnp.int32),
            pltpu.VMEM((HC, CHUNK), jnp.int32),
            pltpu.VMEM((HC, CHUNK), jnp.int32),
            pltpu.VMEM((CHUNK, F), jnp.float32),
            pltpu.VMEM((CHUNK, F), jnp.float32),
            pltpu.VMEM((CHUNK, F), jnp.float32),
            pltpu.VMEM((CHUNK, F), jnp.float32),
            pltpu.SemaphoreType.DMA,
            pltpu.SemaphoreType.DMA,
            pltpu.SemaphoreType.DMA,
            pltpu.SemaphoreType.DMA,
            pltpu.SemaphoreType.DMA,
            pltpu.SemaphoreType.DMA,
            pltpu.SemaphoreType.DMA,
            pltpu.SemaphoreType.DMA,
            pltpu.SemaphoreType.DMA,
            pltpu.SemaphoreType.DMA,
            pltpu.VMEM_SHARED((AR, F), jnp.float32),
        ],
    )
    return fn(g1, g2, row2d, col2d)


def _tc1_body(x_ref, w1_ref, w2_ref, deg_ref, g1_ref, g2_ref, dinv_ref):
    xb = x_ref[...]
    d = deg_ref[0] + deg_ref[1] + 1.0
    dinv = lax.rsqrt(d)
    h1 = jnp.dot(xb, w1_ref[...], preferred_element_type=jnp.float32)
    h2 = jnp.dot(xb, w2_ref[...], preferred_element_type=jnp.float32)
    g1_ref[...] = h1 * dinv
    g2_ref[...] = h2 * dinv
    dinv_ref[...] = dinv


def _tc1_call(x_p, W1, W2, deg3):
    nb = NP // 1024
    return pl.pallas_call(
        _tc1_body,
        grid=(nb,),
        in_specs=[
            pl.BlockSpec((1024, F), lambda i: (i, 0)),
            pl.BlockSpec((F, F), lambda i: (0, 0)),
            pl.BlockSpec((F, F), lambda i: (0, 0)),
            pl.BlockSpec((2, 1024, 1), lambda i: (0, i, 0)),
        ],
        out_specs=[
            pl.BlockSpec((1024, F), lambda i: (i, 0)),
            pl.BlockSpec((1024, F), lambda i: (i, 0)),
            pl.BlockSpec((1024, 1), lambda i: (i, 0)),
        ],
        out_shape=[jax.ShapeDtypeStruct((NP, F), jnp.float32),
                   jax.ShapeDtypeStruct((NP, F), jnp.float32),
                   jax.ShapeDtypeStruct((NP, 1), jnp.float32)],
    )(x_p, W1, W2, deg3)


def _tc2_body(s1_ref, s2_ref, g1_ref, g2_ref, dinv_ref, b1_ref, b2_ref,
              batch_ref, eps_ref, z_ref, mu_ref, lv_ref,
              p1_acc, p2_acc, cnt_acc):
    i = pl.program_id(0)

    @pl.when(i == 0)
    def _():
        p1_acc[...] = jnp.zeros_like(p1_acc)
        p2_acc[...] = jnp.zeros_like(p2_acc)
        cnt_acc[...] = jnp.zeros_like(cnt_acc)

    dinv = dinv_ref[...]
    r1 = jnp.maximum(dinv * (s1_ref[...] + g1_ref[...]) + b1_ref[...], 0.0)
    r2 = jnp.maximum(dinv * (s2_ref[...] + g2_ref[...]) + b2_ref[...], 0.0)

    bt = batch_ref[...]
    seg = lax.broadcasted_iota(jnp.int32, (1024, B), 1)
    oh = jnp.where(bt == seg, 1.0, 0.0).astype(jnp.float32)
    dn = (((0,), (0,)), ((), ()))
    p1_acc[...] += lax.dot_general(oh, r1, dn, preferred_element_type=jnp.float32)
    p2_acc[...] += lax.dot_general(oh, r2, dn, preferred_element_type=jnp.float32)
    cnt_acc[...] += lax.dot_general(oh, jnp.ones((1024, F), jnp.float32), dn,
                                    preferred_element_type=jnp.float32)

    @pl.when(i == pl.num_programs(0) - 1)
    def _():
        cnt = jnp.maximum(cnt_acc[...], 1.0)
        mu = p1_acc[...] / cnt
        lv = p2_acc[...] / cnt
        mu_ref[...] = mu
        lv_ref[...] = lv
        z_ref[...] = eps_ref[...] * jnp.exp(lv) + mu


def _tc2_call(s1, s2, g1, g2, dinv, b1, b2, batch2d, eps):
    nb = NP // 1024
    blk = lambda i: (i, 0)
    return pl.pallas_call(
        _tc2_body,
        grid=(nb,),
        in_specs=[
            pl.BlockSpec((1024, F), blk),
            pl.BlockSpec((1024, F), blk),
            pl.BlockSpec((1024, F), blk),
            pl.BlockSpec((1024, F), blk),
            pl.BlockSpec((1024, 1), blk),
            pl.BlockSpec((1, F), lambda i: (0, 0)),
            pl.BlockSpec((1, F), lambda i: (0, 0)),
            pl.BlockSpec((1024, 1), blk),
            pl.BlockSpec((B, F), lambda i: (0, 0)),
        ],
        out_specs=[
            pl.BlockSpec((B, F), lambda i: (0, 0)),
            pl.BlockSpec((B, F), lambda i: (0, 0)),
            pl.BlockSpec((B, F), lambda i: (0, 0)),
        ],
        out_shape=[jax.ShapeDtypeStruct((B, F), jnp.float32),
                   jax.ShapeDtypeStruct((B, F), jnp.float32),
                   jax.ShapeDtypeStruct((B, F), jnp.float32)],
        scratch_shapes=[
            pltpu.VMEM((B, F), jnp.float32),
            pltpu.VMEM((B, F), jnp.float32),
            pltpu.VMEM((B, F), jnp.float32),
        ],
    )(s1, s2, g1, g2, dinv, b1, b2, batch2d, eps)


def kernel(x, edge_index, batch, W1, b1, W2, b2):
    row = edge_index[0]
    col = edge_index[1]
    pad_idx = (N + (jnp.arange(EP - E, dtype=jnp.int32) % 40))
    row_p = jnp.concatenate([row, pad_idx]).reshape(NCH, CHUNK)
    col_p = jnp.concatenate([col, pad_idx]).reshape(NCH, CHUNK)
    x_p = jnp.pad(x, ((0, NP - N), (0, 0)))
    batch2d = jnp.pad(batch, (0, NP - N), constant_values=B).reshape(NP, 1)
    eps = jax.random.normal(jax.random.key(1), (B, F), dtype=jnp.float32)

    deg2 = _deg_call(col_p)
    deg3 = deg2.reshape(2, NP, 1)
    g1, g2, dinv = _tc1_call(x_p, W1, W2, deg3)
    s1, s2 = _scat_call(g1, g2, row_p, col_p)
    z, mu, lv = _tc2_call(s1, s2, g1, g2, dinv,
                          b1.reshape(1, F), b2.reshape(1, F), batch2d, eps)
    return (z, mu, lv)
